# baseline (speedup 1.0000x reference)
import functools

import jax
import jax.numpy as jnp
from jax import lax
from jax.experimental import pallas as pl
from jax.experimental.pallas import tpu as pltpu
from jax.experimental.pallas import tpu_sc as plsc

NU = 25000
NI = 25000
D = 64
E = 800000
B = 4096

NC = 2
NS = 16
L = 16

RPT = 1568
NPAD = NS * RPT
NT = 2 * NPAD
ITEM_OFF = NPAD
DUMMY = 25080

K = 128
CPT = 400
EPT = CPT * K
EPAD = NS * EPT
BLK = 16
NBLK = CPT // BLK
NCROW = NC * NS * CPT
NBUF = 2

WQ = 14
WR = RPT // WQ

BT = B // (NC * NS)

_mesh = plsc.VectorSubcoreMesh(
    core_axis_name="c", subcore_axis_name="s", num_cores=NC, num_subcores=NS
)
_params = pltpu.CompilerParams(
    use_tc_tiling_on_sc=False, needs_layout_passes=False)


def _rsqrt_newton(x):
    i = lax.bitcast_convert_type(x, jnp.int32)
    y = lax.bitcast_convert_type(jnp.int32(0x5F3759DF) - (i >> 1), jnp.float32)
    for _ in range(3):
        y = y * (1.5 - 0.5 * x * y * y)
    return y


def _deg_body(didx2, e0, dis_out, t0_out, hist, ibuf, ones, dbuf, ebuf,
              ssem, isem, esem0, esem1, osem0, osem1):
    esem = (esem0, esem1)
    osem = (osem0, osem1)
    c = lax.axis_index("c")
    s = lax.axis_index("s")
    crow = (c * NS + s) * CPT

    def init_ones(i, carry):
        ones[pl.ds(i * L, L)] = jnp.full((L,), 1.0, jnp.float32)
        return carry

    lax.fori_loop(0, K // L, init_ones, 0)
    if K % L:
        ones[pl.ds(K - L, L)] = jnp.full((L,), 1.0, jnp.float32)

    def zero_d(i, carry):
        dbuf[pl.ds(i * L, L)] = jnp.zeros((L,), jnp.float32)
        return carry

    lax.fori_loop(0, RPT // L, zero_d, 0)
    pltpu.sync_copy(dbuf, hist.at[pl.ds(s * RPT, RPT)])
    plsc.subcore_barrier()

    def fire_s(bb, k):
        pltpu.async_copy(ones, hist.at[ibuf.at[bb, k]], ssem, add=True)

    def drain_s():
        pltpu.make_async_copy(dis_out.at[pl.ds(0, K)], ones, ssem).wait()

    def fire_i(bnext):
        @pl.when(bnext < NBLK)
        def _():
            pltpu.async_copy(
                didx2.at[pl.ds(crow + bnext * BLK, BLK)],
                ibuf.at[bnext % 2], isem)

    def wait_i():
        pltpu.make_async_copy(
            didx2.at[pl.ds(0, BLK)], ibuf.at[0], isem).wait()

    pltpu.sync_copy(didx2.at[pl.ds(crow, BLK)], ibuf.at[0])
    for k in range(BLK):
        fire_s(0, k)
    fire_i(1)

    def block(b, carry):
        bp = b % 2
        wait_i()
        for _ in range(BLK):
            drain_s()
        for k in range(BLK):
            fire_s(bp, k)
        fire_i(b + 1)
        return carry

    lax.fori_loop(1, NBLK, block, 0)
    for _ in range(BLK):
        drain_s()
    plsc.subcore_barrier()

    pltpu.sync_copy(hist.at[pl.ds(s * RPT, RPT)], dbuf)

    def dis_step(i, carry):
        dg = dbuf[pl.ds(i * L, L)]
        m = dg > 0.0
        y = _rsqrt_newton(jnp.where(m, dg, 1.0))
        dbuf[pl.ds(i * L, L)] = jnp.where(m, y, jnp.zeros((L,), jnp.float32))
        return carry

    lax.fori_loop(0, RPT // L, dis_step, 0)
    gbase = c * NPAD + s * RPT
    pltpu.sync_copy(dbuf, dis_out.at[pl.ds(gbase, RPT)])

    def fire_ein(q):
        pltpu.async_copy(e0.at[pl.ds(gbase + q * WR, WR)],
                         ebuf.at[q % 2], esem[q % 2])

    def wait_ein(p):
        pltpu.make_async_copy(e0.at[pl.ds(0, WR)], ebuf.at[p],
                              esem[p]).wait()

    def fire_eout(q):
        pltpu.async_copy(ebuf.at[q % 2],
                         t0_out.at[pl.ds(gbase + q * WR, WR)], osem[q % 2])

    def wait_eout(p):
        pltpu.make_async_copy(e0.at[pl.ds(0, WR)], ebuf.at[p],
                              osem[p]).wait()

    fire_ein(0)
    for q in range(WQ):
        p = q % 2
        if q + 1 < WQ:
            if q - 1 >= 0:
                wait_eout(1 - p)
            fire_ein(q + 1)
        wait_ein(p)

        def blk_step(b, carry, _q=q, _p=p):
            dvec = dbuf[pl.ds(_q * WR + b * L, L)]
            for rr in range(L):
                r = b * L + rr
                dv = dvec[rr]
                for c4 in range(D // L):
                    sl = pl.ds(c4 * L, L)
                    ebuf[_p, r, sl] = dv * ebuf[_p, r, sl]
            return carry

        lax.fori_loop(0, WR // L, blk_step, 0)
        fire_eout(q)
    wait_eout(WQ % 2)
    wait_eout(1 - WQ % 2)


_deg_fn = functools.partial(
    pl.kernel,
    out_type=(
        jax.ShapeDtypeStruct((NT,), jnp.float32),
        jax.ShapeDtypeStruct((NT, D), jnp.float32),
    ),
    mesh=_mesh,
    compiler_params=_params,
    scratch_types=(
        pltpu.VMEM_SHARED((NPAD,), jnp.float32),
        pltpu.VMEM((2, BLK, K), jnp.int32),
        pltpu.VMEM((K,), jnp.float32),
        pltpu.VMEM((RPT,), jnp.float32),
        pltpu.VMEM((2, WR, D), jnp.float32),
        pltpu.SemaphoreType.DMA,
        pltpu.SemaphoreType.DMA,
        pltpu.SemaphoreType.DMA,
        pltpu.SemaphoreType.DMA,
        pltpu.SemaphoreType.DMA,
        pltpu.SemaphoreType.DMA,
    ),
)(_deg_body)


def _make_layer(emit_t):
    def body(tin, idx, dis, z2, out,
             acc, ibuf, gbuf, dbuf, gsem0, gsem1,
             ssem0, ssem1, isem):
        c = lax.axis_index("c")
        s = lax.axis_index("s")
        crow = (c * NS + s) * CPT
        gsem = (gsem0, gsem1)
        ssem = (ssem0, ssem1)

        pltpu.sync_copy(z2, acc.at[pl.ds(s * RPT, RPT)])

        def fire_g(p, bb, k):
            pltpu.async_copy(tin.at[ibuf.at[bb, k, 0]], gbuf.at[p], gsem[p])

        def fire_s(p, bb, k):
            pltpu.async_copy(
                gbuf.at[p], acc.at[ibuf.at[bb, k, 1]], ssem[p], add=True)

        def wait_g(p):
            pltpu.make_async_copy(
                tin.at[pl.ds(0, K)], gbuf.at[p], gsem[p]).wait()

        def wait_s(p):
            pltpu.make_async_copy(
                tin.at[pl.ds(0, K)], gbuf.at[p], ssem[p]).wait()

        def fire_i(bnext):
            @pl.when(bnext < NBLK)
            def _():
                pltpu.async_copy(
                    idx.at[pl.ds(crow + bnext * BLK, BLK)],
                    ibuf.at[bnext % 2], isem)

        def wait_i():
            pltpu.make_async_copy(
                idx.at[pl.ds(0, BLK)], ibuf.at[0], isem).wait()

        pltpu.sync_copy(idx.at[pl.ds(crow, BLK)], ibuf.at[0])
        plsc.subcore_barrier()
        fire_g(0, 0, 0)
        fire_g(1, 0, 1)
        wait_g(0)
        fire_s(0, 0, 0)
        for k in range(2, BLK):
            p = k % NBUF
            wait_s(p)
            fire_g(p, 0, k)
            wait_g((k - 1) % NBUF)
            fire_s((k - 1) % NBUF, 0, k - 1)
        fire_i(1)

        def block(b, carry):
            bp = b % 2
            bq = 1 - bp
            wait_i()
            for k in range(BLK):
                p = k % NBUF
                pm1 = (k - 1) % NBUF
                wait_s(p)
                fire_g(p, bp, k)
                wait_g(pm1)
                if k == 0:
                    fire_s(pm1, bq, BLK - 1)
                else:
                    fire_s(pm1, bp, k - 1)
            fire_i(b + 1)
            return carry

        lax.fori_loop(1, NBLK, block, 0)
        pl_last = (BLK - 1) % NBUF
        wait_g(pl_last)
        fire_s(pl_last, (NBLK - 1) % 2, BLK - 1)
        wait_s(0)
        wait_s(1)
        plsc.subcore_barrier()

        gbase = c * NPAD + s * RPT
        pltpu.sync_copy(dis.at[pl.ds(gbase, RPT)], dbuf)

        def fire_in(q):
            p = q % NBUF
            pltpu.async_copy(
                acc.at[pl.ds(s * RPT + q * WR, WR)],
                gbuf.at[p, pl.ds(0, WR)], gsem[p])

        def wait_in(p):
            pltpu.make_async_copy(
                tin.at[pl.ds(0, WR)], gbuf.at[p, pl.ds(0, WR)],
                gsem[p]).wait()

        def fire_out(q):
            p = q % NBUF
            pltpu.async_copy(
                gbuf.at[p, pl.ds(0, WR)],
                out.at[pl.ds(gbase + q * WR, WR)], ssem[p])

        def wait_out(p):
            pltpu.make_async_copy(
                tin.at[pl.ds(0, WR)], gbuf.at[p, pl.ds(0, WR)],
                ssem[p]).wait()

        fire_in(0)
        for q in range(WQ):
            p = q % NBUF
            if q + 1 < WQ:
                if q + 1 - NBUF >= 0:
                    wait_out((q + 1) % NBUF)
                fire_in(q + 1)
            wait_in(p)

            def blk_step(b, carry, _q=q, _p=p):
                dvec = dbuf[pl.ds(_q * WR + b * L, L)]
                if emit_t:
                    dvec = dvec * dvec
                for rr in range(L):
                    r = b * L + rr
                    dv = dvec[rr]
                    for c4 in range(D // L):
                        sl = pl.ds(c4 * L, L)
                        gbuf[_p, r, sl] = dv * gbuf[_p, r, sl]
                return carry

            lax.fori_loop(0, WR // L, blk_step, 0)
            fire_out(q)
        for t in range(NBUF):
            wait_out((WQ - NBUF + t) % NBUF)

    return pl.kernel(
        body,
        out_type=jax.ShapeDtypeStruct((NT, D), jnp.float32),
        mesh=_mesh,
        compiler_params=_params,
        scratch_types=(
            pltpu.VMEM_SHARED((NPAD, D), jnp.float32),
            pltpu.VMEM((2, BLK, 2, K), jnp.int32),
            pltpu.VMEM((NBUF, K, D), jnp.float32),
            pltpu.VMEM((RPT,), jnp.float32),
            pltpu.SemaphoreType.DMA,
            pltpu.SemaphoreType.DMA,
            pltpu.SemaphoreType.DMA,
            pltpu.SemaphoreType.DMA,
            pltpu.SemaphoreType.DMA,
        ),
    )


_layer_t = _make_layer(True)
_layer_e = _make_layer(False)


def _final_body(e0, t1, t2, e3, dis, uidx, iidx, scores,
                uib, iib, u0, u1, u2, u3, i0, i1, i2, i3,
                ud, idd, dots, sem):
    c = lax.axis_index("c")
    s = lax.axis_index("s")
    base = (s * NC + c) * BT

    pltpu.sync_copy(uidx.at[pl.ds(base, BT)], uib)
    pltpu.sync_copy(iidx.at[pl.ds(base, BT)], iib)

    pltpu.async_copy(e0.at[uib], u0, sem).wait()
    pltpu.async_copy(t1.at[uib], u1, sem).wait()
    pltpu.async_copy(t2.at[uib], u2, sem).wait()
    pltpu.async_copy(e3.at[uib], u3, sem).wait()
    pltpu.async_copy(e0.at[iib], i0, sem).wait()
    pltpu.async_copy(t1.at[iib], i1, sem).wait()
    pltpu.async_copy(t2.at[iib], i2, sem).wait()
    pltpu.async_copy(e3.at[iib], i3, sem).wait()
    pltpu.async_copy(dis.at[uib], ud, sem).wait()
    pltpu.async_copy(dis.at[iib], idd, sem).wait()

    def inv_step(i, carry):
        du = ud[pl.ds(i * L, L)]
        di = idd[pl.ds(i * L, L)]
        ud[pl.ds(i * L, L)] = jnp.where(du > 0.0, 1.0 / jnp.where(
            du > 0.0, du, 1.0), jnp.zeros((L,), jnp.float32))
        idd[pl.ds(i * L, L)] = jnp.where(di > 0.0, 1.0 / jnp.where(
            di > 0.0, di, 1.0), jnp.zeros((L,), jnp.float32))
        return carry

    lax.fori_loop(0, BT // L, inv_step, 0)

    lane = lax.broadcasted_iota(jnp.int32, (L,), 0)

    def blk(i, carry):
        iuv = ud[pl.ds(i * L, L)]
        iiv = idd[pl.ds(i * L, L)]
        dv = jnp.zeros((L,), jnp.float32)
        for rr in range(L):
            r = i * L + rr
            iu = iuv[rr]
            ii = iiv[rr]
            accv = jnp.zeros((L,), jnp.float32)
            for c4 in range(D // L):
                sl = pl.ds(c4 * L, L)
                su = u0[r, sl] + u3[r, sl] + (u1[r, sl] + u2[r, sl]) * iu
                si = i0[r, sl] + i3[r, sl] + (i1[r, sl] + i2[r, sl]) * ii
                accv = accv + su * si
            dot = jnp.sum(accv) * (1.0 / 16.0)
            dv = jnp.where(lane == rr, dot, dv)
        dots[pl.ds(i * L, L)] = 1.0 / (1.0 + jnp.exp(-dv))
        return carry

    lax.fori_loop(0, BT // L, blk, 0)
    pltpu.sync_copy(dots, scores.at[pl.ds(base, BT)])


_final_fn = functools.partial(
    pl.kernel,
    out_type=jax.ShapeDtypeStruct((B,), jnp.float32),
    mesh=_mesh,
    compiler_params=_params,
    scratch_types=(
        pltpu.VMEM((BT,), jnp.int32),
        pltpu.VMEM((BT,), jnp.int32),
        pltpu.VMEM((BT, D), jnp.float32),
        pltpu.VMEM((BT, D), jnp.float32),
        pltpu.VMEM((BT, D), jnp.float32),
        pltpu.VMEM((BT, D), jnp.float32),
        pltpu.VMEM((BT, D), jnp.float32),
        pltpu.VMEM((BT, D), jnp.float32),
        pltpu.VMEM((BT, D), jnp.float32),
        pltpu.VMEM((BT, D), jnp.float32),
        pltpu.VMEM((BT,), jnp.float32),
        pltpu.VMEM((BT,), jnp.float32),
        pltpu.VMEM((BT,), jnp.float32),
        pltpu.SemaphoreType.DMA,
    ),
)(_final_body)


def kernel(user_table, item_table, edge_index, user, pos):
    eu = edge_index[0].astype(jnp.int32)
    ei = edge_index[1].astype(jnp.int32)

    npad_e = EPAD - E
    pad0 = jnp.zeros((npad_e,), jnp.int32)
    padd = jnp.full((npad_e,), DUMMY, jnp.int32)
    gidx = jnp.concatenate([ei + ITEM_OFF, pad0, eu, pad0])
    didx = jnp.concatenate([eu, padd, ei, padd])
    didx2 = didx.reshape(NCROW, K)
    idx = jnp.stack([gidx.reshape(NCROW, K), didx2], axis=1)

    zu = jnp.zeros((NPAD - NU, D), jnp.float32)
    zi = jnp.zeros((NPAD - NI, D), jnp.float32)
    e0 = jnp.concatenate([user_table, zu, item_table, zi], axis=0)
    z2 = jnp.zeros((RPT, D), jnp.float32)

    dis, t0 = _deg_fn(didx2, e0)
    t1 = _layer_t(t0, idx, dis, z2)
    t2 = _layer_t(t1, idx, dis, z2)
    e3 = _layer_e(t2, idx, dis, z2)

    iw = pos.astype(jnp.int32) + ITEM_OFF
    return _final_fn(e0, t1, t2, e3, dis, user.astype(jnp.int32), iw)


# speedup vs baseline: 2.2459x; 2.2459x over previous
import functools

import jax
import jax.numpy as jnp
from jax import lax
from jax.experimental import pallas as pl
from jax.experimental.pallas import tpu as pltpu
from jax.experimental.pallas import tpu_sc as plsc

NU = 25000
NI = 25000
D = 64
E = 800000
B = 4096

NC = 2
NS = 16
L = 16

RPT = 1568
NPAD = NS * RPT
NT = 2 * NPAD
ITEM_OFF = NPAD
DUMMY = 25080

K = 128
CPT = 392
EPT = CPT * K
EPAD = NS * EPT
BLK = 8
NBLK = CPT // BLK
NCROW = NC * NS * CPT
NBUF = 2

WQ = 14
WR = RPT // WQ

BT = B // (NC * NS)

_mesh = plsc.VectorSubcoreMesh(
    core_axis_name="c", subcore_axis_name="s", num_cores=NC, num_subcores=NS
)
_params = pltpu.CompilerParams(
    use_tc_tiling_on_sc=False, needs_layout_passes=False)


def _rsqrt_newton(x):
    i = lax.bitcast_convert_type(x, jnp.int32)
    y = lax.bitcast_convert_type(jnp.int32(0x5F3759DF) - (i >> 1), jnp.float32)
    for _ in range(3):
        y = y * (1.5 - 0.5 * x * y * y)
    return y


def _deg_body(didx2, e0, dis_out, t0_out, hist, ibuf, ones, dbuf, ebuf,
              ssem, isem, esem0, esem1, osem0, osem1):
    esem = (esem0, esem1)
    osem = (osem0, osem1)
    c = lax.axis_index("c")
    s = lax.axis_index("s")
    crow = (c * NS + s) * CPT

    def init_ones(i, carry):
        ones[pl.ds(i * L, L)] = jnp.full((L,), 1.0, jnp.float32)
        return carry

    lax.fori_loop(0, K // L, init_ones, 0)
    if K % L:
        ones[pl.ds(K - L, L)] = jnp.full((L,), 1.0, jnp.float32)

    def zero_d(i, carry):
        dbuf[pl.ds(i * L, L)] = jnp.zeros((L,), jnp.float32)
        return carry

    lax.fori_loop(0, RPT // L, zero_d, 0)
    pltpu.sync_copy(dbuf, hist.at[pl.ds(s * RPT, RPT)])
    plsc.subcore_barrier()

    def fire_s(bb, k):
        pltpu.async_copy(ones, hist.at[ibuf.at[bb, k]], ssem, add=True)

    def drain_s():
        pltpu.make_async_copy(dis_out.at[pl.ds(0, K)], ones, ssem).wait()

    def fire_i(bnext):
        @pl.when(bnext < NBLK)
        def _():
            pltpu.async_copy(
                didx2.at[pl.ds(crow + bnext * BLK, BLK)],
                ibuf.at[bnext % 2], isem)

    def wait_i():
        pltpu.make_async_copy(
            didx2.at[pl.ds(0, BLK)], ibuf.at[0], isem).wait()

    pltpu.sync_copy(didx2.at[pl.ds(crow, BLK)], ibuf.at[0])
    for k in range(BLK):
        fire_s(0, k)
    fire_i(1)

    def block(b, carry):
        bp = b % 2
        wait_i()
        for _ in range(BLK):
            drain_s()
        for k in range(BLK):
            fire_s(bp, k)
        fire_i(b + 1)
        return carry

    lax.fori_loop(1, NBLK, block, 0)
    for _ in range(BLK):
        drain_s()
    plsc.subcore_barrier()

    pltpu.sync_copy(hist.at[pl.ds(s * RPT, RPT)], dbuf)

    def dis_step(i, carry):
        dg = dbuf[pl.ds(i * L, L)]
        m = dg > 0.0
        y = _rsqrt_newton(jnp.where(m, dg, 1.0))
        dbuf[pl.ds(i * L, L)] = jnp.where(m, y, jnp.zeros((L,), jnp.float32))
        return carry

    lax.fori_loop(0, RPT // L, dis_step, 0)
    gbase = c * NPAD + s * RPT
    pltpu.sync_copy(dbuf, dis_out.at[pl.ds(gbase, RPT)])

    def fire_ein(q):
        pltpu.async_copy(e0.at[pl.ds(gbase + q * WR, WR)],
                         ebuf.at[q % 2], esem[q % 2])

    def wait_ein(p):
        pltpu.make_async_copy(e0.at[pl.ds(0, WR)], ebuf.at[p],
                              esem[p]).wait()

    def fire_eout(q):
        pltpu.async_copy(ebuf.at[q % 2],
                         t0_out.at[pl.ds(gbase + q * WR, WR)], osem[q % 2])

    def wait_eout(p):
        pltpu.make_async_copy(e0.at[pl.ds(0, WR)], ebuf.at[p],
                              osem[p]).wait()

    fire_ein(0)
    for q in range(WQ):
        p = q % 2
        if q + 1 < WQ:
            if q - 1 >= 0:
                wait_eout(1 - p)
            fire_ein(q + 1)
        wait_ein(p)

        def blk_step(b, carry, _q=q, _p=p):
            dvec = dbuf[pl.ds(_q * WR + b * L, L)]
            for rr in range(L):
                r = b * L + rr
                dv = dvec[rr]
                for c4 in range(D // L):
                    sl = pl.ds(c4 * L, L)
                    ebuf[_p, r, sl] = dv * ebuf[_p, r, sl]
            return carry

        lax.fori_loop(0, WR // L, blk_step, 0)
        fire_eout(q)
    wait_eout(WQ % 2)
    wait_eout(1 - WQ % 2)


_deg_fn = functools.partial(
    pl.kernel,
    out_type=(
        jax.ShapeDtypeStruct((NT,), jnp.float32),
        jax.ShapeDtypeStruct((NT, D), jnp.float32),
    ),
    mesh=_mesh,
    compiler_params=_params,
    scratch_types=(
        pltpu.VMEM_SHARED((NPAD,), jnp.float32),
        pltpu.VMEM((2, BLK, K), jnp.int32),
        pltpu.VMEM((K,), jnp.float32),
        pltpu.VMEM((RPT,), jnp.float32),
        pltpu.VMEM((2, WR, D), jnp.float32),
        pltpu.SemaphoreType.DMA,
        pltpu.SemaphoreType.DMA,
        pltpu.SemaphoreType.DMA,
        pltpu.SemaphoreType.DMA,
        pltpu.SemaphoreType.DMA,
        pltpu.SemaphoreType.DMA,
    ),
)(_deg_body)


def _make_layer(emit_t):
    def body(tin, idx, dis, z2, out,
             acc, ibuf, gbuf, dbuf, gsem0, gsem1,
             ssem0, ssem1, isem):
        c = lax.axis_index("c")
        s = lax.axis_index("s")
        crow = (c * NS + s) * CPT
        gsem = (gsem0, gsem1)
        ssem = (ssem0, ssem1)

        pltpu.sync_copy(z2, acc.at[pl.ds(s * RPT, RPT)])

        def fire_g(p, bb, k):
            pltpu.async_copy(tin.at[ibuf.at[bb, k, 0]], gbuf.at[p], gsem[p])

        def fire_s(p, bb, k):
            pltpu.async_copy(
                gbuf.at[p], acc.at[ibuf.at[bb, k, 1]], ssem[p], add=True)

        def wait_g(p):
            pltpu.make_async_copy(
                tin.at[pl.ds(0, K)], gbuf.at[p], gsem[p]).wait()

        def wait_s(p):
            pltpu.make_async_copy(
                tin.at[pl.ds(0, K)], gbuf.at[p], ssem[p]).wait()

        def fire_i(bnext):
            @pl.when(bnext < NBLK)
            def _():
                pltpu.async_copy(
                    idx.at[pl.ds(crow + bnext * BLK, BLK)],
                    ibuf.at[bnext % 2], isem)

        def wait_i():
            pltpu.make_async_copy(
                idx.at[pl.ds(0, BLK)], ibuf.at[0], isem).wait()

        pltpu.sync_copy(idx.at[pl.ds(crow, BLK)], ibuf.at[0])
        plsc.subcore_barrier()
        fire_g(0, 0, 0)
        fire_g(1, 0, 1)
        wait_g(0)
        fire_s(0, 0, 0)
        for k in range(2, BLK):
            p = k % NBUF
            wait_s(p)
            fire_g(p, 0, k)
            wait_g((k - 1) % NBUF)
            fire_s((k - 1) % NBUF, 0, k - 1)
        fire_i(1)

        def block(b, carry):
            bp = b % 2
            bq = 1 - bp
            wait_i()
            for k in range(BLK):
                p = k % NBUF
                pm1 = (k - 1) % NBUF
                wait_s(p)
                fire_g(p, bp, k)
                wait_g(pm1)
                if k == 0:
                    fire_s(pm1, bq, BLK - 1)
                else:
                    fire_s(pm1, bp, k - 1)
            fire_i(b + 1)
            return carry

        lax.fori_loop(1, NBLK, block, 0)
        pl_last = (BLK - 1) % NBUF
        wait_g(pl_last)
        fire_s(pl_last, (NBLK - 1) % 2, BLK - 1)
        wait_s(0)
        wait_s(1)
        plsc.subcore_barrier()

        gbase = c * NPAD + s * RPT
        pltpu.sync_copy(dis.at[pl.ds(gbase, RPT)], dbuf)

        def fire_in(q):
            p = q % NBUF
            pltpu.async_copy(
                acc.at[pl.ds(s * RPT + q * WR, WR)],
                gbuf.at[p, pl.ds(0, WR)], gsem[p])

        def wait_in(p):
            pltpu.make_async_copy(
                tin.at[pl.ds(0, WR)], gbuf.at[p, pl.ds(0, WR)],
                gsem[p]).wait()

        def fire_out(q):
            p = q % NBUF
            pltpu.async_copy(
                gbuf.at[p, pl.ds(0, WR)],
                out.at[pl.ds(gbase + q * WR, WR)], ssem[p])

        def wait_out(p):
            pltpu.make_async_copy(
                tin.at[pl.ds(0, WR)], gbuf.at[p, pl.ds(0, WR)],
                ssem[p]).wait()

        fire_in(0)
        for q in range(WQ):
            p = q % NBUF
            if q + 1 < WQ:
                if q + 1 - NBUF >= 0:
                    wait_out((q + 1) % NBUF)
                fire_in(q + 1)
            wait_in(p)

            def blk_step(b, carry, _q=q, _p=p):
                dvec = dbuf[pl.ds(_q * WR + b * L, L)]
                if emit_t:
                    dvec = dvec * dvec
                for rr in range(L):
                    r = b * L + rr
                    dv = dvec[rr]
                    for c4 in range(D // L):
                        sl = pl.ds(c4 * L, L)
                        gbuf[_p, r, sl] = dv * gbuf[_p, r, sl]
                return carry

            lax.fori_loop(0, WR // L, blk_step, 0)
            fire_out(q)
        for t in range(NBUF):
            wait_out((WQ - NBUF + t) % NBUF)

    return pl.kernel(
        body,
        out_type=jax.ShapeDtypeStruct((NT, D), jnp.float32),
        mesh=_mesh,
        compiler_params=_params,
        scratch_types=(
            pltpu.VMEM_SHARED((NPAD, D), jnp.float32),
            pltpu.VMEM((2, BLK, 2, K), jnp.int32),
            pltpu.VMEM((NBUF, K, D), jnp.float32),
            pltpu.VMEM((RPT,), jnp.float32),
            pltpu.SemaphoreType.DMA,
            pltpu.SemaphoreType.DMA,
            pltpu.SemaphoreType.DMA,
            pltpu.SemaphoreType.DMA,
            pltpu.SemaphoreType.DMA,
        ),
    )


_layer_t = _make_layer(True)
_layer_e = _make_layer(False)


def _final_body(e0, t1, t2, e3, dis, uidx, iidx, scores,
                uib, iib, u0, u1, u2, u3, i0, i1, i2, i3,
                ud, idd, dots, sem):
    c = lax.axis_index("c")
    s = lax.axis_index("s")
    base = (s * NC + c) * BT

    pltpu.sync_copy(uidx.at[pl.ds(base, BT)], uib)
    pltpu.sync_copy(iidx.at[pl.ds(base, BT)], iib)

    pltpu.async_copy(e0.at[uib], u0, sem).wait()
    pltpu.async_copy(t1.at[uib], u1, sem).wait()
    pltpu.async_copy(t2.at[uib], u2, sem).wait()
    pltpu.async_copy(e3.at[uib], u3, sem).wait()
    pltpu.async_copy(e0.at[iib], i0, sem).wait()
    pltpu.async_copy(t1.at[iib], i1, sem).wait()
    pltpu.async_copy(t2.at[iib], i2, sem).wait()
    pltpu.async_copy(e3.at[iib], i3, sem).wait()
    pltpu.async_copy(dis.at[uib], ud, sem).wait()
    pltpu.async_copy(dis.at[iib], idd, sem).wait()

    def inv_step(i, carry):
        du = ud[pl.ds(i * L, L)]
        di = idd[pl.ds(i * L, L)]
        ud[pl.ds(i * L, L)] = jnp.where(du > 0.0, 1.0 / jnp.where(
            du > 0.0, du, 1.0), jnp.zeros((L,), jnp.float32))
        idd[pl.ds(i * L, L)] = jnp.where(di > 0.0, 1.0 / jnp.where(
            di > 0.0, di, 1.0), jnp.zeros((L,), jnp.float32))
        return carry

    lax.fori_loop(0, BT // L, inv_step, 0)

    lane = lax.broadcasted_iota(jnp.int32, (L,), 0)

    def blk(i, carry):
        iuv = ud[pl.ds(i * L, L)]
        iiv = idd[pl.ds(i * L, L)]
        dv = jnp.zeros((L,), jnp.float32)
        for rr in range(L):
            r = i * L + rr
            iu = iuv[rr]
            ii = iiv[rr]
            accv = jnp.zeros((L,), jnp.float32)
            for c4 in range(D // L):
                sl = pl.ds(c4 * L, L)
                su = u0[r, sl] + u3[r, sl] + (u1[r, sl] + u2[r, sl]) * iu
                si = i0[r, sl] + i3[r, sl] + (i1[r, sl] + i2[r, sl]) * ii
                accv = accv + su * si
            dot = jnp.sum(accv) * (1.0 / 16.0)
            dv = jnp.where(lane == rr, dot, dv)
        dots[pl.ds(i * L, L)] = 1.0 / (1.0 + jnp.exp(-dv))
        return carry

    lax.fori_loop(0, BT // L, blk, 0)
    pltpu.sync_copy(dots, scores.at[pl.ds(base, BT)])


_final_fn = functools.partial(
    pl.kernel,
    out_type=jax.ShapeDtypeStruct((B,), jnp.float32),
    mesh=_mesh,
    compiler_params=_params,
    scratch_types=(
        pltpu.VMEM((BT,), jnp.int32),
        pltpu.VMEM((BT,), jnp.int32),
        pltpu.VMEM((BT, D), jnp.float32),
        pltpu.VMEM((BT, D), jnp.float32),
        pltpu.VMEM((BT, D), jnp.float32),
        pltpu.VMEM((BT, D), jnp.float32),
        pltpu.VMEM((BT, D), jnp.float32),
        pltpu.VMEM((BT, D), jnp.float32),
        pltpu.VMEM((BT, D), jnp.float32),
        pltpu.VMEM((BT, D), jnp.float32),
        pltpu.VMEM((BT,), jnp.float32),
        pltpu.VMEM((BT,), jnp.float32),
        pltpu.VMEM((BT,), jnp.float32),
        pltpu.SemaphoreType.DMA,
    ),
)(_final_body)


def kernel(user_table, item_table, edge_index, user, pos):
    eu = edge_index[0].astype(jnp.int32)
    ei = edge_index[1].astype(jnp.int32)

    npad_e = EPAD - E
    pad0 = jnp.zeros((npad_e,), jnp.int32)
    padd = NU + (jnp.arange(npad_e, dtype=jnp.int32) % (NPAD - NU - 1))
    gidx = jnp.concatenate([ei + ITEM_OFF, pad0, eu, pad0])
    didx = jnp.concatenate([eu, padd, ei, padd])
    didx2 = didx.reshape(NCROW, K)
    idx = jnp.stack([gidx.reshape(NCROW, K), didx2], axis=1)

    zu = jnp.zeros((NPAD - NU, D), jnp.float32)
    zi = jnp.zeros((NPAD - NI, D), jnp.float32)
    e0 = jnp.concatenate([user_table, zu, item_table, zi], axis=0)
    z2 = jnp.zeros((RPT, D), jnp.float32)

    dis, t0 = _deg_fn(didx2, e0)
    t1 = _layer_t(t0, idx, dis, z2)
    t2 = _layer_t(t1, idx, dis, z2)
    e3 = _layer_e(t2, idx, dis, z2)

    iw = pos.astype(jnp.int32) + ITEM_OFF
    return _final_fn(e0, t1, t2, e3, dis, user.astype(jnp.int32), iw)
